# baseline (speedup 1.0000x reference)
import jax
import jax.numpy as jnp
from jax.experimental import pallas as pl


def _body(p_ref, o_ref):
    o_ref[...] = jnp.broadcast_to(p_ref[...][None], o_ref.shape)


def kernel(inputs, params):
    n, t, d = inputs.shape
    bt = 1024
    return pl.pallas_call(
        _body,
        grid=(t // bt,),
        in_specs=[pl.BlockSpec((bt, d), lambda i: (i, 0))],
        out_specs=pl.BlockSpec((n, bt, d), lambda i: (0, i, 0)),
        out_shape=jax.ShapeDtypeStruct((n, t, d), params.dtype),
    )(params)
